# baseline (speedup 1.0000x reference)
import functools

import jax
import jax.numpy as jnp
from jax import lax
from jax.experimental import pallas as pl
from jax.experimental.pallas import tpu as pltpu
from jax.experimental.pallas import tpu_sc as plsc

B = 16384
N_SPARSE = 26
N_DENSE = 13
N_FIELDS = N_SPARSE + N_DENSE
VOCAB = 100000
DIM = 16

NC = 2
NSUB = 16
NW = NC * NSUB
ROWS_PER_W = B // NW
CHUNK_B = 16
CHUNKS = ROWS_PER_W // CHUNK_B
K = CHUNK_B * N_SPARSE
KD = CHUNK_B * N_DENSE
_SUBS = [(o, min(128, K - o)) for o in range(0, K, 128)]

_mesh = plsc.VectorSubcoreMesh(core_axis_name="c", subcore_axis_name="s")


@functools.partial(
    pl.kernel,
    mesh=_mesh,
    out_type=jax.ShapeDtypeStruct((B, N_FIELDS * DIM), jnp.float32),
    scratch_types=[
        pltpu.VMEM((K,), jnp.int32),
        pltpu.VMEM((K,), jnp.int32),
        pltpu.VMEM((K, 128), jnp.float32),
        pltpu.VMEM((CHUNK_B, N_FIELDS * DIM), jnp.float32),
        pltpu.VMEM((KD,), jnp.float32),
        pltpu.VMEM((N_DENSE, DIM), jnp.float32),
        pltpu.SemaphoreType.DMA,
    ],
    compiler_params=pltpu.CompilerParams(
        needs_layout_passes=False, use_tc_tiling_on_sc=True
    ),
)
def _sc_embed(idx_hbm, vals_hbm, demb_hbm, table_hbm, out_hbm,
              idx_v, q_v, rows_v, comb_v, vals_v, demb_v, sem):
    wid = lax.axis_index("s") * NC + lax.axis_index("c")

    pltpu.sync_copy(demb_hbm, demb_v)
    iota = lax.iota(jnp.int32, 16)
    d_consts = [jnp.full((16,), d, jnp.int32) for d in range(DIM)]

    def chunk_body(c, _):
        b0 = pl.multiple_of((wid * CHUNKS + c) * CHUNK_B, CHUNK_B)
        pltpu.sync_copy(
            idx_hbm.at[pl.ds(pl.multiple_of(b0 * N_SPARSE, 8), K)], idx_v)
        pltpu.sync_copy(
            vals_hbm.at[pl.ds(pl.multiple_of(b0 * N_DENSE, 8), KD)], vals_v)

        def qs(g, _):
            q_v[pl.ds(g * 16, 16)] = idx_v[pl.ds(g * 16, 16)] >> 3
            return 0
        lax.fori_loop(0, K // 16, qs, 0)

        gathers = [
            pltpu.async_copy(
                table_hbm.at[q_v.at[pl.ds(o, n)]],
                rows_v.at[pl.ds(o, n), :],
                sem,
            )
            for (o, n) in _SUBS
        ]

        for j in range(N_DENSE):
            ev = demb_v[j, :]
            flat = iota * N_DENSE + j
            val = plsc.load_gather(vals_v, [flat])
            col0 = N_SPARSE * DIM + j * DIM
            for d in range(DIM):
                plsc.store_scatter(
                    comb_v, [iota, d_consts[d] + col0], val * ev[d]
                )

        for gth in gathers:
            gth.wait()

        def ext(g, _):
            k_vec = g * 16 + iota
            lane0 = (idx_v[pl.ds(g * 16, 16)] & 7) << 4
            b_vec = k_vec // N_SPARSE
            col0 = (k_vec - b_vec * N_SPARSE) << 4
            for d in range(DIM):
                v = plsc.load_gather(rows_v, [k_vec, lane0 + d_consts[d]])
                plsc.store_scatter(comb_v, [b_vec, col0 + d_consts[d]], v)
            return 0
        lax.fori_loop(0, K // 16, ext, 0)

        pltpu.sync_copy(comb_v, out_hbm.at[pl.ds(b0, CHUNK_B), :])
        return 0

    lax.fori_loop(0, CHUNKS, chunk_body, 0)


def kernel(x, sparse_tables, dense_tables):
    x = x.astype(jnp.int32)
    offs = (jnp.arange(N_SPARSE, dtype=jnp.int32) * VOCAB)[None, :]
    flat_idx = (x[:, :N_SPARSE] + offs).reshape(-1)
    vals = x[:, N_SPARSE:].astype(jnp.float32).reshape(-1)
    table = sparse_tables.reshape(N_SPARSE * VOCAB // 8, 8 * DIM)
    demb = dense_tables.reshape(N_DENSE, DIM)
    out = _sc_embed(flat_idx, vals, demb, table)
    return out


# speedup vs baseline: 1.1610x; 1.1610x over previous
import functools

import jax
import jax.numpy as jnp
import numpy as np
from jax import lax
from jax.experimental import pallas as pl
from jax.experimental.pallas import tpu as pltpu
from jax.experimental.pallas import tpu_sc as plsc

B = 16384
N_SPARSE = 26
N_DENSE = 13
N_FIELDS = N_SPARSE + N_DENSE
VOCAB = 100000
DIM = 16

NC = 2
NSUB = 16
NW = NC * NSUB
ROWS_PER_W = B // NW
CHUNK = 128
CHUNKS = ROWS_PER_W // CHUNK
SP_SUB = CHUNK * N_SPARSE // 128
DN_SUB = CHUNK * N_DENSE // 128

TOTAL_CHUNKS = B // CHUNK
_n_sp = np.arange(B * N_SPARSE, dtype=np.int32)
_SIDX_SP = ((_n_sp // N_SPARSE) * N_FIELDS + _n_sp % N_SPARSE).reshape(
    TOTAL_CHUNKS, SP_SUB, 128
)
_n_dn = np.arange(B * N_DENSE, dtype=np.int32)
_SIDX_DN = (
    (_n_dn // N_DENSE) * N_FIELDS + N_SPARSE + _n_dn % N_DENSE
).reshape(TOTAL_CHUNKS, DN_SUB, 128)

_mesh = plsc.VectorSubcoreMesh(core_axis_name="c", subcore_axis_name="s")


@functools.partial(
    pl.kernel,
    mesh=_mesh,
    out_type=jax.ShapeDtypeStruct((B * N_FIELDS, DIM), jnp.float32),
    scratch_types=[
        pltpu.VMEM((CHUNK * N_SPARSE,), jnp.int32),
        pltpu.VMEM((SP_SUB, 128), jnp.int32),
        pltpu.VMEM((DN_SUB, 128), jnp.int32),
        pltpu.VMEM((CHUNK * N_SPARSE, DIM), jnp.float32),
        pltpu.VMEM((CHUNK * N_DENSE, DIM), jnp.float32),
        pltpu.VMEM((DN_SUB, 128), jnp.float32),
        pltpu.VMEM((N_DENSE, DIM), jnp.float32),
        pltpu.SemaphoreType.DMA,
        pltpu.SemaphoreType.DMA,
    ],
    compiler_params=pltpu.CompilerParams(
        needs_layout_passes=False, use_tc_tiling_on_sc=False
    ),
)
def _sc_embed(idx_hbm, vals_hbm, demb_hbm, table_hbm, sidx_sp_hbm, sidx_dn_hbm,
              out_hbm, idx_v, ssp_v, sdn_v, rows_v, dense_v, vals_v, demb_v,
              sem_in, sem_out):
    wid = lax.axis_index("s") * NC + lax.axis_index("c")

    pltpu.sync_copy(demb_hbm, demb_v)
    iota = lax.iota(jnp.int32, 16)
    demb_bc = [
        [demb_v[j, :][d] for d in range(DIM)] for j in range(N_DENSE)
    ]
    d_consts = [jnp.full((16,), d, jnp.int32) for d in range(DIM)]
    iota13 = iota * N_DENSE

    for c in range(CHUNKS):
        b0 = wid * ROWS_PER_W + c * CHUNK
        t = wid * CHUNKS + c
        pltpu.sync_copy(idx_hbm.at[pl.ds(b0 * N_SPARSE, CHUNK * N_SPARSE)], idx_v)
        pltpu.sync_copy(sidx_sp_hbm.at[t], ssp_v)
        pltpu.sync_copy(sidx_dn_hbm.at[t], sdn_v)
        pltpu.sync_copy(vals_hbm.at[t], vals_v)

        gathers = [
            pltpu.async_copy(
                table_hbm.at[idx_v.at[pl.ds(i * 128, 128)]],
                rows_v.at[pl.ds(i * 128, 128), :],
                sem_in,
            )
            for i in range(SP_SUB)
        ]

        for j in range(N_DENSE):
            def dense_body(g, _, j=j):
                flat = g * (16 * N_DENSE) + iota13 + j
                val = plsc.load_gather(
                    vals_v, [flat >> 7, flat & 127]
                )
                for d in range(DIM):
                    plsc.store_scatter(
                        dense_v, [flat, d_consts[d]], val * demb_bc[j][d]
                    )
                return 0
            lax.fori_loop(0, CHUNK // 16, dense_body, 0)

        dn_scatters = [
            pltpu.async_copy(
                dense_v.at[pl.ds(i * 128, 128), :],
                out_hbm.at[sdn_v.at[i]],
                sem_out,
            )
            for i in range(DN_SUB)
        ]

        for g in gathers:
            g.wait()

        sp_scatters = [
            pltpu.async_copy(
                rows_v.at[pl.ds(i * 128, 128), :],
                out_hbm.at[ssp_v.at[i]],
                sem_out,
            )
            for i in range(SP_SUB)
        ]
        for s in dn_scatters + sp_scatters:
            s.wait()


def kernel(x, sparse_tables, dense_tables):
    x = x.astype(jnp.int32)
    offs = (jnp.arange(N_SPARSE, dtype=jnp.int32) * VOCAB)[None, :]
    flat_idx = (x[:, :N_SPARSE] + offs).reshape(-1)
    vals = x[:, N_SPARSE:].astype(jnp.float32).reshape(
        TOTAL_CHUNKS, DN_SUB, 128
    )
    table = sparse_tables.reshape(N_SPARSE * VOCAB, DIM)
    demb = dense_tables.reshape(N_DENSE, DIM)
    out = _sc_embed(flat_idx, vals, demb, table, _SIDX_SP, _SIDX_DN)
    return out.reshape(B, N_FIELDS * DIM)
